# Initial kernel scaffold; baseline (speedup 1.0000x reference)
import jax
import jax.numpy as jnp
from jax.experimental import pallas as pl


def kernel(target_features, source_features, select_mat, W, b):
    raise NotImplementedError("write your pallas kernel here")


# speedup vs baseline: 99.0726x; 99.0726x over previous
import jax
import jax.numpy as jnp
from jax.experimental import pallas as pl

NT = 512
NS = 256
FEA = 512
GATE = 128
BT = 64


def _msdn_kernel(tf_ref, sf_ref, sel_ref, w1_ref, w2_ref, b_ref, out_ref):
    tfb = jnp.maximum(tf_ref[...], 0.0)
    sf = sf_ref[...]
    sfr = jnp.maximum(sf, 0.0)
    a = jnp.dot(tfb, w1_ref[...], preferred_element_type=jnp.float32)
    bm = jnp.dot(sfr, w2_ref[...], preferred_element_type=jnp.float32)
    logits = a[:, None, :] + (bm + b_ref[...])[None, :, :]
    gate = jnp.mean(jax.nn.sigmoid(logits), axis=-1)
    mask = (sel_ref[...] > 0.0).astype(jnp.float32)
    mg = mask * gate
    counts = jnp.sum(mask, axis=1, keepdims=True)
    seg = jnp.dot(mg, sf, preferred_element_type=jnp.float32)
    out_ref[...] = jnp.where(counts > 0.0, seg / jnp.maximum(counts, 1.0), 0.0)


@jax.jit
def kernel(target_features, source_features, select_mat, W, b):
    w1 = W[:, :FEA].T
    w2 = W[:, FEA:].T
    b2 = b.reshape(1, GATE)
    grid = NT // BT
    return pl.pallas_call(
        _msdn_kernel,
        grid=(grid,),
        in_specs=[
            pl.BlockSpec((BT, FEA), lambda i: (i, 0)),
            pl.BlockSpec((NS, FEA), lambda i: (0, 0)),
            pl.BlockSpec((BT, NS), lambda i: (i, 0)),
            pl.BlockSpec((FEA, GATE), lambda i: (0, 0)),
            pl.BlockSpec((FEA, GATE), lambda i: (0, 0)),
            pl.BlockSpec((1, GATE), lambda i: (0, 0)),
        ],
        out_specs=pl.BlockSpec((BT, FEA), lambda i: (i, 0)),
        out_shape=jax.ShapeDtypeStruct((NT, FEA), jnp.float32),
    )(target_features, source_features, select_mat, w1, w2, b2)


# speedup vs baseline: 109.6016x; 1.1063x over previous
import jax
import jax.numpy as jnp
from jax.experimental import pallas as pl

NT = 512
NS = 256
FEA = 512
GATE = 128
BT = 128


def _msdn_kernel(tf_ref, sf_ref, sel_ref, w1_ref, w2_ref, b_ref, out_ref):
    tfb = jnp.maximum(tf_ref[...], 0.0)
    sf = sf_ref[...]
    sfr = jnp.maximum(sf, 0.0)
    a = 0.5 * jnp.dot(tfb, w1_ref[...], preferred_element_type=jnp.float32)
    bm = 0.5 * (jnp.dot(sfr, w2_ref[...], preferred_element_type=jnp.float32)
                + b_ref[...])
    h = jnp.tanh(a[:, None, :] + bm[None, :, :])
    gate = 0.5 + jnp.sum(h, axis=-1) * (0.5 / GATE)
    mask = (sel_ref[...] > 0.0).astype(jnp.float32)
    mg = mask * gate
    counts = jnp.sum(mask, axis=1, keepdims=True)
    seg = jnp.dot(mg, sf, preferred_element_type=jnp.float32)
    out_ref[...] = jnp.where(counts > 0.0, seg / jnp.maximum(counts, 1.0), 0.0)


@jax.jit
def kernel(target_features, source_features, select_mat, W, b):
    w1 = W[:, :FEA].T
    w2 = W[:, FEA:].T
    b2 = b.reshape(1, GATE)
    grid = NT // BT
    return pl.pallas_call(
        _msdn_kernel,
        grid=(grid,),
        in_specs=[
            pl.BlockSpec((BT, FEA), lambda i: (i, 0)),
            pl.BlockSpec((NS, FEA), lambda i: (0, 0)),
            pl.BlockSpec((BT, NS), lambda i: (i, 0)),
            pl.BlockSpec((FEA, GATE), lambda i: (0, 0)),
            pl.BlockSpec((FEA, GATE), lambda i: (0, 0)),
            pl.BlockSpec((1, GATE), lambda i: (0, 0)),
        ],
        out_specs=pl.BlockSpec((BT, FEA), lambda i: (i, 0)),
        out_shape=jax.ShapeDtypeStruct((NT, FEA), jnp.float32),
    )(target_features, source_features, select_mat, w1, w2, b2)


# speedup vs baseline: 123.5578x; 1.1273x over previous
import jax
import jax.numpy as jnp
from jax.experimental import pallas as pl

NT = 512
NS = 256
FEA = 512
GATE = 128
BT = 128


def _msdn_kernel(tf_ref, sf_ref, sel_ref, w_ref, b_ref, out_ref):
    tfb = jnp.maximum(tf_ref[...], 0.0)
    sf = sf_ref[...]
    sfr = jnp.maximum(sf, 0.0)
    w = w_ref[...]
    a = 0.5 * jax.lax.dot_general(
        tfb, w[:, :FEA], (((1,), (1,)), ((), ())),
        preferred_element_type=jnp.float32)
    bm = 0.5 * (jax.lax.dot_general(
        sfr, w[:, FEA:], (((1,), (1,)), ((), ())),
        preferred_element_type=jnp.float32) + b_ref[...])
    h = jnp.tanh(a[:, None, :] + bm[None, :, :])
    gate = 0.5 + jnp.sum(h, axis=-1) * (0.5 / GATE)
    mask = (sel_ref[...] > 0.0).astype(jnp.float32)
    mg = mask * gate
    counts = jnp.sum(mask, axis=1, keepdims=True)
    seg = jnp.dot(mg, sf, preferred_element_type=jnp.float32)
    out_ref[...] = jnp.where(counts > 0.0, seg / jnp.maximum(counts, 1.0), 0.0)


@jax.jit
def kernel(target_features, source_features, select_mat, W, b):
    b2 = b.reshape(1, GATE)
    grid = NT // BT
    return pl.pallas_call(
        _msdn_kernel,
        grid=(grid,),
        in_specs=[
            pl.BlockSpec((BT, FEA), lambda i: (i, 0)),
            pl.BlockSpec((NS, FEA), lambda i: (0, 0)),
            pl.BlockSpec((BT, NS), lambda i: (i, 0)),
            pl.BlockSpec((GATE, 2 * FEA), lambda i: (0, 0)),
            pl.BlockSpec((1, GATE), lambda i: (0, 0)),
        ],
        out_specs=pl.BlockSpec((BT, FEA), lambda i: (i, 0)),
        out_shape=jax.ShapeDtypeStruct((NT, FEA), jnp.float32),
    )(target_features, source_features, select_mat, W, b2)


# speedup vs baseline: 127.9419x; 1.0355x over previous
import jax
import jax.numpy as jnp
from jax.experimental import pallas as pl

NT = 512
NS = 256
FEA = 512
GATE = 128
BT = 256


def _msdn_kernel(tf_ref, sf_ref, sel_ref, w_ref, b_ref, out_ref):
    tfb = jnp.maximum(tf_ref[...], 0.0)
    sf = sf_ref[...]
    sfr = jnp.maximum(sf, 0.0)
    w = w_ref[...]
    a = 0.5 * jax.lax.dot_general(
        tfb, w[:, :FEA], (((1,), (1,)), ((), ())),
        preferred_element_type=jnp.float32)
    bm = 0.5 * (jax.lax.dot_general(
        sfr, w[:, FEA:], (((1,), (1,)), ((), ())),
        preferred_element_type=jnp.float32) + b_ref[...])
    h = jnp.tanh(a[:, None, :] + bm[None, :, :])
    gate = 0.5 + jnp.sum(h, axis=-1) * (0.5 / GATE)
    mask = (sel_ref[...] > 0.0).astype(jnp.float32)
    mg = mask * gate
    counts = jnp.sum(mask, axis=1, keepdims=True)
    seg = jnp.dot(mg, sf, preferred_element_type=jnp.float32)
    out_ref[...] = jnp.where(counts > 0.0, seg / jnp.maximum(counts, 1.0), 0.0)


@jax.jit
def kernel(target_features, source_features, select_mat, W, b):
    b2 = b.reshape(1, GATE)
    grid = NT // BT
    return pl.pallas_call(
        _msdn_kernel,
        grid=(grid,),
        in_specs=[
            pl.BlockSpec((BT, FEA), lambda i: (i, 0)),
            pl.BlockSpec((NS, FEA), lambda i: (0, 0)),
            pl.BlockSpec((BT, NS), lambda i: (i, 0)),
            pl.BlockSpec((GATE, 2 * FEA), lambda i: (0, 0)),
            pl.BlockSpec((1, GATE), lambda i: (0, 0)),
        ],
        out_specs=pl.BlockSpec((BT, FEA), lambda i: (i, 0)),
        out_shape=jax.ShapeDtypeStruct((NT, FEA), jnp.float32),
    )(target_features, source_features, select_mat, W, b2)


# speedup vs baseline: 128.4766x; 1.0042x over previous
import jax
import jax.numpy as jnp
from jax.experimental import pallas as pl

NT = 512
NS = 256
FEA = 512
GATE = 128
BT = 512


def _msdn_kernel(tf_ref, sf_ref, sel_ref, w_ref, b_ref, out_ref):
    tfb = jnp.maximum(tf_ref[...], 0.0)
    sf = sf_ref[...]
    sfr = jnp.maximum(sf, 0.0)
    w = w_ref[...]
    a = 0.5 * jax.lax.dot_general(
        tfb, w[:, :FEA], (((1,), (1,)), ((), ())),
        preferred_element_type=jnp.float32)
    bm = 0.5 * (jax.lax.dot_general(
        sfr, w[:, FEA:], (((1,), (1,)), ((), ())),
        preferred_element_type=jnp.float32) + b_ref[...])
    h = jnp.tanh(a[:, None, :] + bm[None, :, :])
    gate = 0.5 + jnp.sum(h, axis=-1) * (0.5 / GATE)
    mask = (sel_ref[...] > 0.0).astype(jnp.float32)
    mg = mask * gate
    counts = jnp.sum(mask, axis=1, keepdims=True)
    seg = jnp.dot(mg, sf, preferred_element_type=jnp.float32)
    out_ref[...] = jnp.where(counts > 0.0, seg / jnp.maximum(counts, 1.0), 0.0)


@jax.jit
def kernel(target_features, source_features, select_mat, W, b):
    b2 = b.reshape(1, GATE)
    grid = NT // BT
    return pl.pallas_call(
        _msdn_kernel,
        grid=(grid,),
        in_specs=[
            pl.BlockSpec((BT, FEA), lambda i: (i, 0)),
            pl.BlockSpec((NS, FEA), lambda i: (0, 0)),
            pl.BlockSpec((BT, NS), lambda i: (i, 0)),
            pl.BlockSpec((GATE, 2 * FEA), lambda i: (0, 0)),
            pl.BlockSpec((1, GATE), lambda i: (0, 0)),
        ],
        out_specs=pl.BlockSpec((BT, FEA), lambda i: (i, 0)),
        out_shape=jax.ShapeDtypeStruct((NT, FEA), jnp.float32),
    )(target_features, source_features, select_mat, W, b2)
